# baseline (speedup 1.0000x reference)
import functools

import jax
import jax.numpy as jnp
from jax import lax
from jax.experimental import pallas as pl
from jax.experimental.pallas import tpu as pltpu
from jax.experimental.pallas import tpu_sc as plsc

C = 192
H = W = 384
GH = GW = 384
N = GH * GW
NPIX = H * W
NC, NS = 2, 16
NW = NC * NS
PTS_PER_W = N // NW
CHUNK = 32
NCHUNK = PTS_PER_W // CHUNK
CG = C // 16


def _sc_sample(table, idx4, w4):
    mesh = plsc.VectorSubcoreMesh(core_axis_name="c", subcore_axis_name="s")

    @functools.partial(
        pl.kernel,
        out_type=jax.ShapeDtypeStruct((N, C), jnp.float32),
        mesh=mesh,
        scratch_types=[
            pltpu.VMEM((4, PTS_PER_W), jnp.int32),
            pltpu.VMEM((4, PTS_PER_W), jnp.float32),
            pltpu.VMEM((2, 4, CHUNK, C), jnp.float32),
            pltpu.VMEM((2, CHUNK, C), jnp.float32),
            pltpu.SemaphoreType.DMA,
            pltpu.SemaphoreType.DMA,
        ],
        compiler_params=pltpu.CompilerParams(use_tc_tiling_on_sc=False),
    )
    def k(table_hbm, idx_hbm, w_hbm, out_hbm,
          stage_i, stage_w, rows_v, out_v, sem_g, sem_o):
        wid = lax.axis_index("s") * NC + lax.axis_index("c")
        wbase = wid * PTS_PER_W
        pltpu.sync_copy(idx_hbm.at[:, pl.ds(wbase, PTS_PER_W)], stage_i)
        pltpu.sync_copy(w_hbm.at[:, pl.ds(wbase, PTS_PER_W)], stage_w)

        def fire(ci):
            par = lax.rem(ci, 2)
            for j in range(4):
                pltpu.async_copy(
                    table_hbm.at[stage_i.at[j, pl.ds(ci * CHUNK, CHUNK)]],
                    rows_v.at[par, j], sem_g)

        fire(0)

        def chunk_body(ci, carry):
            par = lax.rem(ci, 2)

            @pl.when(ci + 1 < NCHUNK)
            def _():
                fire(ci + 1)

            for j in range(4):
                pltpu.make_async_copy(
                    table_hbm.at[stage_i.at[j, pl.ds(ci * CHUNK, CHUNK)]],
                    rows_v.at[par, j], sem_g).wait()

            @pl.when(ci >= 2)
            def _():
                pltpu.make_async_copy(
                    out_v.at[par], out_hbm.at[pl.ds(0, CHUNK)], sem_o).wait()

            for b in range(CHUNK // 16):
                wv = [stage_w[j, pl.ds(ci * CHUNK + b * 16, 16)]
                      for j in range(4)]
                for t in range(16):
                    i = b * 16 + t
                    w00, w01, w10, w11 = wv[0][t], wv[1][t], wv[2][t], wv[3][t]
                    for g in range(CG):
                        s = pl.ds(g * 16, 16)
                        out_v[par, i, s] = (rows_v[par, 0, i, s] * w00
                                            + rows_v[par, 1, i, s] * w01
                                            + rows_v[par, 2, i, s] * w10
                                            + rows_v[par, 3, i, s] * w11)

            pltpu.async_copy(
                out_v.at[par], out_hbm.at[pl.ds(wbase + ci * CHUNK, CHUNK)],
                sem_o)
            return carry

        lax.fori_loop(0, NCHUNK, chunk_body, 0)
        pltpu.make_async_copy(
            out_v.at[0], out_hbm.at[pl.ds(0, CHUNK)], sem_o).wait()
        pltpu.make_async_copy(
            out_v.at[1], out_hbm.at[pl.ds(0, CHUNK)], sem_o).wait()

    return k(table, idx4, w4)


def kernel(grid, matrix):
    x = grid[0, :, :, 0].reshape(-1)
    y = grid[0, :, :, 1].reshape(-1)
    ix = jnp.clip(((x + 1.0) * W - 1.0) / 2.0, 0.0, W - 1.0)
    iy = jnp.clip(((y + 1.0) * H - 1.0) / 2.0, 0.0, H - 1.0)
    ix0f = jnp.floor(ix)
    iy0f = jnp.floor(iy)
    wx = ix - ix0f
    wy = iy - iy0f
    ix0 = jnp.clip(ix0f.astype(jnp.int32), 0, W - 1)
    ix1 = jnp.clip(ix0 + 1, 0, W - 1)
    iy0 = jnp.clip(iy0f.astype(jnp.int32), 0, H - 1)
    iy1 = jnp.clip(iy0 + 1, 0, H - 1)
    idx4 = jnp.stack([iy0 * W + ix0, iy0 * W + ix1,
                      iy1 * W + ix0, iy1 * W + ix1])
    w4 = jnp.stack([(1.0 - wy) * (1.0 - wx), (1.0 - wy) * wx,
                    wy * (1.0 - wx), wy * wx])
    table = matrix.reshape(C, NPIX).T
    out_flat = _sc_sample(table, idx4, w4)
    return out_flat.T.reshape(1, C, GH, GW)


# speedup vs baseline: 1.1837x; 1.1837x over previous
import functools

import jax
import jax.numpy as jnp
from jax import lax
from jax.experimental import pallas as pl
from jax.experimental.pallas import tpu as pltpu
from jax.experimental.pallas import tpu_sc as plsc

C = 192
H = W = 384
GH = GW = 384
N = GH * GW
NPIX = H * W
NC, NS = 2, 16
NW = NC * NS
PTS_PER_W = N // NW
CHUNK = 32
NCHUNK = PTS_PER_W // CHUNK
GL = 4 * CHUNK
CG = C // 16


def _sc_sample(table, idxc, wc):
    mesh = plsc.VectorSubcoreMesh(core_axis_name="c", subcore_axis_name="s")

    @functools.partial(
        pl.kernel,
        out_type=jax.ShapeDtypeStruct((N, C), jnp.float32),
        mesh=mesh,
        scratch_types=[
            pltpu.VMEM((NCHUNK, GL), jnp.int32),
            pltpu.VMEM((NCHUNK, GL), jnp.float32),
            pltpu.VMEM((2, GL, C), jnp.float32),
            pltpu.VMEM((2, CHUNK, C), jnp.float32),
            pltpu.SemaphoreType.DMA,
            pltpu.SemaphoreType.DMA,
        ],
        compiler_params=pltpu.CompilerParams(use_tc_tiling_on_sc=False),
    )
    def k(table_hbm, idx_hbm, w_hbm, out_hbm,
          stage_i, stage_w, rows_v, out_v, sem_g, sem_o):
        wid = lax.axis_index("s") * NC + lax.axis_index("c")
        wbase = wid * PTS_PER_W
        pltpu.sync_copy(idx_hbm.at[pl.ds(wid * NCHUNK, NCHUNK)], stage_i)
        pltpu.sync_copy(w_hbm.at[pl.ds(wid * NCHUNK, NCHUNK)], stage_w)

        def fire(ci, par):
            pltpu.async_copy(
                table_hbm.at[stage_i.at[ci]], rows_v.at[par], sem_g)

        def wait_gather(par):
            pltpu.make_async_copy(
                table_hbm.at[stage_i.at[0]], rows_v.at[par], sem_g).wait()

        def wait_write():
            pltpu.make_async_copy(
                out_v.at[0], out_hbm.at[pl.ds(0, CHUNK)], sem_o).wait()

        def step(ci, par):
            @pl.when(ci + 1 < NCHUNK)
            def _():
                fire(ci + 1, 1 - par)

            wait_gather(par)

            @pl.when(ci >= 2)
            def _():
                wait_write()

            for b in range(CHUNK // 16):
                wv = [stage_w[ci, pl.ds(j * CHUNK + b * 16, 16)]
                      for j in range(4)]
                for t in range(16):
                    i = b * 16 + t
                    w00, w01, w10, w11 = wv[0][t], wv[1][t], wv[2][t], wv[3][t]
                    for g in range(CG):
                        s = pl.ds(g * 16, 16)
                        out_v[par, i, s] = (
                            rows_v[par, 0 * CHUNK + i, s] * w00
                            + rows_v[par, 1 * CHUNK + i, s] * w01
                            + rows_v[par, 2 * CHUNK + i, s] * w10
                            + rows_v[par, 3 * CHUNK + i, s] * w11)

            pltpu.async_copy(
                out_v.at[par], out_hbm.at[pl.ds(wbase + ci * CHUNK, CHUNK)],
                sem_o)

        fire(0, 0)

        def pair_body(it, carry):
            step(2 * it, 0)
            step(2 * it + 1, 1)
            return carry

        lax.fori_loop(0, NCHUNK // 2, pair_body, 0)
        wait_write()
        wait_write()

    return k(table, idxc, wc)


def kernel(grid, matrix):
    x = grid[0, :, :, 0].reshape(-1)
    y = grid[0, :, :, 1].reshape(-1)
    ix = jnp.clip(((x + 1.0) * W - 1.0) / 2.0, 0.0, W - 1.0)
    iy = jnp.clip(((y + 1.0) * H - 1.0) / 2.0, 0.0, H - 1.0)
    ix0f = jnp.floor(ix)
    iy0f = jnp.floor(iy)
    wx = ix - ix0f
    wy = iy - iy0f
    ix0 = jnp.clip(ix0f.astype(jnp.int32), 0, W - 1)
    ix1 = jnp.clip(ix0 + 1, 0, W - 1)
    iy0 = jnp.clip(iy0f.astype(jnp.int32), 0, H - 1)
    iy1 = jnp.clip(iy0 + 1, 0, H - 1)
    idx4 = jnp.stack([iy0 * W + ix0, iy0 * W + ix1,
                      iy1 * W + ix0, iy1 * W + ix1])
    w4 = jnp.stack([(1.0 - wy) * (1.0 - wx), (1.0 - wy) * wx,
                    wy * (1.0 - wx), wy * wx])
    idxc = idx4.reshape(4, N // CHUNK, CHUNK).transpose(1, 0, 2).reshape(
        N // CHUNK, GL)
    wc = w4.reshape(4, N // CHUNK, CHUNK).transpose(1, 0, 2).reshape(
        N // CHUNK, GL)
    table = matrix.reshape(C, NPIX).T
    out_flat = _sc_sample(table, idxc, wc)
    return out_flat.T.reshape(1, C, GH, GW)


# speedup vs baseline: 1.8374x; 1.5523x over previous
import functools

import jax
import jax.numpy as jnp
from jax import lax
from jax.experimental import pallas as pl
from jax.experimental.pallas import tpu as pltpu
from jax.experimental.pallas import tpu_sc as plsc

C = 192
H = W = 384
GH = GW = 384
N = GH * GW
NPIX = H * W
NC, NS = 2, 16
NW = NC * NS
PTS_PER_W = N // NW
CHUNK = 32
NCHUNK = PTS_PER_W // CHUNK
GL = 4 * CHUNK
CG = C // 16


def _sc_sample(table, idxc, wc):
    mesh = plsc.VectorSubcoreMesh(core_axis_name="c", subcore_axis_name="s")

    @functools.partial(
        pl.kernel,
        out_type=jax.ShapeDtypeStruct((N, C), jnp.float32),
        mesh=mesh,
        scratch_types=[
            pltpu.VMEM((NCHUNK, GL), jnp.int32),
            pltpu.VMEM((NCHUNK, GL), jnp.float32),
            pltpu.VMEM((2, GL, C), jnp.float32),
            pltpu.VMEM((2, CHUNK, C), jnp.float32),
            pltpu.SemaphoreType.DMA,
            pltpu.SemaphoreType.DMA,
        ],
        compiler_params=pltpu.CompilerParams(use_tc_tiling_on_sc=False,
                                             needs_layout_passes=False),
    )
    def k(table_hbm, idx_hbm, w_hbm, out_hbm,
          stage_i, stage_w, rows_v, out_v, sem_g, sem_o):
        wid = lax.axis_index("s") * NC + lax.axis_index("c")
        wbase = wid * PTS_PER_W
        pltpu.sync_copy(idx_hbm.at[pl.ds(wid * NCHUNK, NCHUNK)], stage_i)
        pltpu.sync_copy(w_hbm.at[pl.ds(wid * NCHUNK, NCHUNK)], stage_w)

        def fire(ci, par):
            pltpu.async_copy(
                table_hbm.at[stage_i.at[ci]], rows_v.at[par], sem_g)

        def wait_gather(par):
            pltpu.make_async_copy(
                table_hbm.at[stage_i.at[0]], rows_v.at[par], sem_g).wait()

        def wait_write():
            pltpu.make_async_copy(
                out_v.at[0], out_hbm.at[pl.ds(0, CHUNK)], sem_o).wait()

        def step(ci, par):
            @pl.when(ci + 1 < NCHUNK)
            def _():
                fire(ci + 1, 1 - par)

            wait_gather(par)

            @pl.when(ci >= 2)
            def _():
                wait_write()

            ci16 = jnp.full((16,), ci, jnp.int32)

            @plsc.parallel_loop(0, CHUNK, unroll=2)
            def pt_body(i):
                w = [plsc.load_gather(stage_w,
                                      [ci16, jnp.full((16,), j * CHUNK + i,
                                                      jnp.int32)])
                     for j in range(4)]
                for g in range(CG):
                    s = pl.ds(g * 16, 16)
                    out_v[par, i, s] = (
                        rows_v[par, 0 * CHUNK + i, s] * w[0]
                        + rows_v[par, 1 * CHUNK + i, s] * w[1]
                        + rows_v[par, 2 * CHUNK + i, s] * w[2]
                        + rows_v[par, 3 * CHUNK + i, s] * w[3])

            pltpu.async_copy(
                out_v.at[par], out_hbm.at[pl.ds(wbase + ci * CHUNK, CHUNK)],
                sem_o)

        fire(0, 0)

        def pair_body(it, carry):
            step(2 * it, 0)
            step(2 * it + 1, 1)
            return carry

        lax.fori_loop(0, NCHUNK // 2, pair_body, 0)
        wait_write()
        wait_write()

    return k(table, idxc, wc)


def kernel(grid, matrix):
    x = grid[0, :, :, 0].reshape(-1)
    y = grid[0, :, :, 1].reshape(-1)
    ix = jnp.clip(((x + 1.0) * W - 1.0) / 2.0, 0.0, W - 1.0)
    iy = jnp.clip(((y + 1.0) * H - 1.0) / 2.0, 0.0, H - 1.0)
    ix0f = jnp.floor(ix)
    iy0f = jnp.floor(iy)
    wx = ix - ix0f
    wy = iy - iy0f
    ix0 = jnp.clip(ix0f.astype(jnp.int32), 0, W - 1)
    ix1 = jnp.clip(ix0 + 1, 0, W - 1)
    iy0 = jnp.clip(iy0f.astype(jnp.int32), 0, H - 1)
    iy1 = jnp.clip(iy0 + 1, 0, H - 1)
    idx4 = jnp.stack([iy0 * W + ix0, iy0 * W + ix1,
                      iy1 * W + ix0, iy1 * W + ix1])
    w4 = jnp.stack([(1.0 - wy) * (1.0 - wx), (1.0 - wy) * wx,
                    wy * (1.0 - wx), wy * wx])
    idxc = idx4.reshape(4, N // CHUNK, CHUNK).transpose(1, 0, 2).reshape(
        N // CHUNK, GL)
    wc = w4.reshape(4, N // CHUNK, CHUNK).transpose(1, 0, 2).reshape(
        N // CHUNK, GL)
    table = matrix.reshape(C, NPIX).T
    out_flat = _sc_sample(table, idxc, wc)
    return out_flat.T.reshape(1, C, GH, GW)
